# baseline (device time: 67291 ns/iter reference)
import jax
import jax.numpy as jnp
from jax import lax
from jax.experimental import pallas as pl
from jax.experimental.pallas import tpu as pltpu

N_DEV = 8
M = 1024
N_TOT = 4096
N_CHUNK = N_TOT // N_DEV

ROW0 = (0, 344, 680)
ROWS = (344, 336, 344)


def _xor(a, b):
    return a + b - 2 * a * b


def _id3(vx, vy, vz):
    return 4 * vz + 2 * vy + _xor(vx, vy)


def kernel(x):
    def body(x_ref, out_ref, r1_0, r2_0, r3_0, r1_1, r2_1, r3_1, r1_2, r2_2,
             r3_2, xk_0, xk_1, xk_2, xs_0, xs_1, xs_2, ssems, rsems, csems,
             dsems):
        p = lax.axis_index("i")
        z = p // 4
        pid = p % 4
        yb = pid // 2
        u = pid % 2
        mx = _xor(u, yb)
        my = yb
        mz = z

        qx = _id3(1 - mx, my, mz)
        qy = _id3(mx, 1 - my, mz)
        qz = _id3(mx, my, 1 - mz)

        xks = (xk_0, xk_1, xk_2)
        xss = (xs_0, xs_1, xs_2)

        barrier_sem = pltpu.get_barrier_semaphore()
        for q in (qx, qy, qz):
            pl.semaphore_signal(
                barrier_sem, inc=1, device_id=(q,),
                device_id_type=pl.DeviceIdType.MESH,
            )
        pl.semaphore_wait(barrier_sem, 3)

        flows = (
            dict(m=(mx, my, mz),
                 idf=lambda v1, v2, v3: _id3(v1, v2, v3),
                 q=(qx, qy, qz), r1=r1_0, r2=r2_0, r3=r3_0),
            dict(m=(my, mz, mx),
                 idf=lambda v1, v2, v3: _id3(v3, v1, v2),
                 q=(qy, qz, qx), r1=r1_1, r2=r2_1, r3=r3_1),
            dict(m=(mz, mx, my),
                 idf=lambda v1, v2, v3: _id3(v2, v3, v1),
                 q=(qz, qx, qy), r1=r1_2, r2=r2_2, r3=r3_2),
        )

        def xsrc(f, j):
            return x_ref.at[0, pl.ds(ROW0[f], ROWS[f]), pl.ds(j * N_CHUNK, N_CHUNK)]

        def mkc(f, t2, t3):
            fl = flows[f]
            m1, m2, m3 = fl["m"]
            j = fl["idf"](m1, _xor(t2, m2), _xor(t3, m3))
            return pltpu.make_async_copy(
                xsrc(f, j), xks[f].at[2 * t2 + t3], csems.at[f, 2 * t2 + t3]
            )

        def mks(f, t2, t3):
            fl = flows[f]
            m1, m2, m3 = fl["m"]
            j = fl["idf"](1 - m1, _xor(t2, m2), _xor(t3, m3))
            return pltpu.make_async_copy(
                xsrc(f, j), xss[f].at[2 * t2 + t3], dsems.at[f, 2 * t2 + t3]
            )

        def mk1(f, t2, t3):
            fl = flows[f]
            return pltpu.make_async_remote_copy(
                src_ref=xss[f].at[2 * t2 + t3],
                dst_ref=fl["r1"].at[2 * t2 + t3],
                send_sem=ssems.at[f, 2 * t2 + t3],
                recv_sem=rsems.at[f, 2 * t2 + t3],
                device_id=(fl["q"][0],),
                device_id_type=pl.DeviceIdType.MESH,
            )

        def mk2(f, t3):
            fl = flows[f]
            return pltpu.make_async_remote_copy(
                src_ref=fl["r1"].at[2 + t3],
                dst_ref=fl["r2"].at[t3],
                send_sem=ssems.at[f, 4 + t3],
                recv_sem=rsems.at[f, 4 + t3],
                device_id=(fl["q"][1],),
                device_id_type=pl.DeviceIdType.MESH,
            )

        def mk3(f):
            fl = flows[f]
            return pltpu.make_async_remote_copy(
                src_ref=fl["r2"].at[1],
                dst_ref=fl["r3"],
                send_sem=ssems.at[f, 6],
                recv_sem=rsems.at[f, 6],
                device_id=(fl["q"][2],),
                device_id_type=pl.DeviceIdType.MESH,
            )

        ORDER = ((1, 1), (1, 0), (0, 1), (0, 0))

        def absorb1(f, t2, t3):
            fl = flows[f]
            s = 2 * t2 + t3
            mk1(f, t2, t3).wait_recv()
            mkc(f, t2, t3).wait()
            fl["r1"][s] = fl["r1"][s] + xks[f][s]

        for t2, t3 in ORDER:
            for f in range(3):
                mks(f, t2, t3).start()
        for t2, t3 in ORDER:
            for f in range(3):
                mkc(f, t2, t3).start()
        for t2, t3 in ORDER:
            for f in range(3):
                mks(f, t2, t3).wait()
                mk1(f, t2, t3).start()

        for f in range(3):
            absorb1(f, 1, 1)
            mk2(f, 1).start()
        for f in range(3):
            absorb1(f, 1, 0)
            mk2(f, 0).start()
        for f in range(3):
            absorb1(f, 0, 1)
        for f in range(3):
            fl = flows[f]
            mk2(f, 1).wait_recv()
            fl["r2"][1] = fl["r2"][1] + fl["r1"][1]
            mk3(f).start()
        for f in range(3):
            absorb1(f, 0, 0)
        for f in range(3):
            fl = flows[f]
            mk2(f, 0).wait_recv()
            fl["r2"][0] = fl["r2"][0] + fl["r1"][0]
        for f in range(3):
            fl = flows[f]
            mk3(f).wait_recv()
            out_ref[pl.ds(ROW0[f], ROWS[f]), :] = fl["r2"][0] + fl["r3"][:, :]

        for f in range(3):
            for t2 in range(2):
                for t3 in range(2):
                    mk1(f, t2, t3).wait_send()
            mk2(f, 0).wait_send()
            mk2(f, 1).wait_send()
            mk3(f).wait_send()

    scratch = []
    for f in range(3):
        scratch.append(pltpu.VMEM((4, ROWS[f], N_CHUNK), jnp.float32))
        scratch.append(pltpu.VMEM((2, ROWS[f], N_CHUNK), jnp.float32))
        scratch.append(pltpu.VMEM((ROWS[f], N_CHUNK), jnp.float32))
    for f in range(3):
        scratch.append(pltpu.VMEM((4, ROWS[f], N_CHUNK), jnp.float32))
    for f in range(3):
        scratch.append(pltpu.VMEM((4, ROWS[f], N_CHUNK), jnp.float32))
    scratch.append(pltpu.SemaphoreType.DMA((3, 7)))
    scratch.append(pltpu.SemaphoreType.DMA((3, 7)))
    scratch.append(pltpu.SemaphoreType.DMA((3, 4)))
    scratch.append(pltpu.SemaphoreType.DMA((3, 4)))

    return pl.pallas_call(
        body,
        out_shape=jax.ShapeDtypeStruct((M, N_CHUNK), jnp.float32),
        in_specs=[pl.BlockSpec(memory_space=pl.ANY)],
        out_specs=pl.BlockSpec(memory_space=pltpu.VMEM),
        scratch_shapes=scratch,
        compiler_params=pltpu.CompilerParams(collective_id=0),
    )(x)


# device time: 65501 ns/iter; 1.0273x vs baseline; 1.0273x over previous
import jax
import jax.numpy as jnp
from jax import lax
from jax.experimental import pallas as pl
from jax.experimental.pallas import tpu as pltpu

N_DEV = 8
M = 1024
N_TOT = 4096
N_CHUNK = N_TOT // N_DEV

ROW0 = (0, 344, 680)
ROWS = (344, 336, 344)


def _xor(a, b):
    return a + b - 2 * a * b


def _id3(vx, vy, vz):
    return 4 * vz + 2 * vy + _xor(vx, vy)


def kernel(x):
    def body(x_ref, out_ref, r1_0, r2_0, r3_0, r1_1, r2_1, r3_1, r1_2, r2_2,
             r3_2, ssems, rsems):
        p = lax.axis_index("i")
        z = p // 4
        pid = p % 4
        yb = pid // 2
        u = pid % 2
        mx = _xor(u, yb)
        my = yb
        mz = z

        qx = _id3(1 - mx, my, mz)
        qy = _id3(mx, 1 - my, mz)
        qz = _id3(mx, my, 1 - mz)

        barrier_sem = pltpu.get_barrier_semaphore()
        for q in (qx, qy, qz):
            pl.semaphore_signal(
                barrier_sem, inc=1, device_id=(q,),
                device_id_type=pl.DeviceIdType.MESH,
            )
        pl.semaphore_wait(barrier_sem, 3)

        flows = (
            dict(m=(mx, my, mz),
                 idf=lambda v1, v2, v3: _id3(v1, v2, v3),
                 q=(qx, qy, qz), r1=r1_0, r2=r2_0, r3=r3_0),
            dict(m=(my, mz, mx),
                 idf=lambda v1, v2, v3: _id3(v3, v1, v2),
                 q=(qy, qz, qx), r1=r1_1, r2=r2_1, r3=r3_1),
            dict(m=(mz, mx, my),
                 idf=lambda v1, v2, v3: _id3(v2, v3, v1),
                 q=(qz, qx, qy), r1=r1_2, r2=r2_2, r3=r3_2),
        )

        def xslice(f, j):
            return x_ref[0, pl.ds(ROW0[f], ROWS[f]), pl.ds(j * N_CHUNK, N_CHUNK)]

        def xsrc(f, j):
            return x_ref.at[0, pl.ds(ROW0[f], ROWS[f]), pl.ds(j * N_CHUNK, N_CHUNK)]

        def mk1(f, t2, t3):
            fl = flows[f]
            m1, m2, m3 = fl["m"]
            j = fl["idf"](1 - m1, _xor(t2, m2), _xor(t3, m3))
            return pltpu.make_async_remote_copy(
                src_ref=xsrc(f, j),
                dst_ref=fl["r1"].at[2 * t2 + t3],
                send_sem=ssems.at[f, 2 * t2 + t3],
                recv_sem=rsems.at[f, 2 * t2 + t3],
                device_id=(fl["q"][0],),
                device_id_type=pl.DeviceIdType.MESH,
            )

        def mk2(f, t3):
            fl = flows[f]
            return pltpu.make_async_remote_copy(
                src_ref=fl["r1"].at[2 + t3],
                dst_ref=fl["r2"].at[t3],
                send_sem=ssems.at[f, 4 + t3],
                recv_sem=rsems.at[f, 4 + t3],
                device_id=(fl["q"][1],),
                device_id_type=pl.DeviceIdType.MESH,
            )

        def mk3(f):
            fl = flows[f]
            return pltpu.make_async_remote_copy(
                src_ref=fl["r2"].at[1],
                dst_ref=fl["r3"],
                send_sem=ssems.at[f, 6],
                recv_sem=rsems.at[f, 6],
                device_id=(fl["q"][2],),
                device_id_type=pl.DeviceIdType.MESH,
            )

        ORDER = ((1, 1), (1, 0), (0, 1), (0, 0))

        def absorb1(f, t2, t3):
            fl = flows[f]
            m1, m2, m3 = fl["m"]
            mk1(f, t2, t3).wait_recv()
            j = fl["idf"](m1, _xor(t2, m2), _xor(t3, m3))
            s = 2 * t2 + t3
            fl["r1"][s] = fl["r1"][s] + xslice(f, j)

        for t2, t3 in ORDER:
            for f in range(3):
                mk1(f, t2, t3).start()

        for f in range(3):
            absorb1(f, 1, 1)
            mk2(f, 1).start()
        for f in range(3):
            absorb1(f, 1, 0)
            mk2(f, 0).start()
        for f in range(3):
            absorb1(f, 0, 1)
        for f in range(3):
            fl = flows[f]
            mk2(f, 1).wait_recv()
            fl["r2"][1] = fl["r2"][1] + fl["r1"][1]
            mk3(f).start()
        for f in range(3):
            absorb1(f, 0, 0)
        for f in range(3):
            fl = flows[f]
            mk2(f, 0).wait_recv()
            fl["r2"][0] = fl["r2"][0] + fl["r1"][0]
        for f in range(3):
            fl = flows[f]
            mk3(f).wait_recv()
            out_ref[pl.ds(ROW0[f], ROWS[f]), :] = fl["r2"][0] + fl["r3"][:, :]

        for f in range(3):
            for t2 in range(2):
                for t3 in range(2):
                    mk1(f, t2, t3).wait_send()
            mk2(f, 0).wait_send()
            mk2(f, 1).wait_send()
            mk3(f).wait_send()

    scratch = []
    for f in range(3):
        scratch.append(pltpu.VMEM((4, ROWS[f], N_CHUNK), jnp.float32))
        scratch.append(pltpu.VMEM((2, ROWS[f], N_CHUNK), jnp.float32))
        scratch.append(pltpu.VMEM((ROWS[f], N_CHUNK), jnp.float32))
    scratch.append(pltpu.SemaphoreType.DMA((3, 7)))
    scratch.append(pltpu.SemaphoreType.DMA((3, 7)))

    return pl.pallas_call(
        body,
        out_shape=jax.ShapeDtypeStruct((M, N_CHUNK), jnp.float32),
        in_specs=[pl.BlockSpec(memory_space=pltpu.VMEM)],
        out_specs=pl.BlockSpec(memory_space=pltpu.VMEM),
        scratch_shapes=scratch,
        compiler_params=pltpu.CompilerParams(collective_id=0),
    )(x)
